# baseline (speedup 1.0000x reference)
import functools

import jax
import jax.numpy as jnp
from jax import lax
from jax.experimental import pallas as pl
from jax.experimental.pallas import tpu as pltpu
from jax.experimental.pallas import tpu_sc as plsc

N = 10000
NE = 320000
D = 128

NC, NS = 2, 16
NW = NC * NS
NPAD = 10240
RPT = NPAD // NS
NE_PAD = 327680
EPT = NE_PAD // NW
CH = 128
NCHUNK = EPT // CH

_MESH = plsc.VectorSubcoreMesh(
    core_axis_name="c", subcore_axis_name="s", num_cores=NC, num_subcores=NS
)


@functools.partial(
    pl.kernel,
    out_type=jax.ShapeDtypeStruct((NC * NPAD, D), jnp.float32),
    mesh=_MESH,
    scratch_types=[
        pltpu.VMEM((NCHUNK, CH), jnp.int32),
        pltpu.VMEM((CH, D), jnp.float32),
        pltpu.VMEM_SHARED((NPAD, D), jnp.float32),
        pltpu.SemaphoreType.DMA,
    ],
)
def _sc_degree(src_hbm, zdeg_hbm, ones_hbm, out_hbm, idx_v, ones_v, acc_sh, sem):
    cid = lax.axis_index("c")
    sid = lax.axis_index("s")
    wid = sid * NC + cid
    pltpu.sync_copy(src_hbm.at[pl.ds(wid * NCHUNK, NCHUNK)], idx_v)
    pltpu.sync_copy(ones_hbm, ones_v)
    pltpu.sync_copy(zdeg_hbm, acc_sh.at[pl.ds(sid * RPT, RPT)])
    plsc.subcore_barrier()

    def fire(ci, carry):
        pltpu.async_copy(ones_v, acc_sh.at[idx_v.at[ci]], sem, add=True)
        return carry

    lax.fori_loop(0, NCHUNK, fire, 0)

    def drain(ci, carry):
        pltpu.make_async_copy(ones_v, acc_sh.at[idx_v.at[0]], sem).wait()
        return carry

    lax.fori_loop(0, NCHUNK, drain, 0)
    plsc.subcore_barrier()
    pltpu.sync_copy(
        acc_sh.at[pl.ds(sid * RPT, RPT)],
        out_hbm.at[pl.ds(cid * NPAD + sid * RPT, RPT)],
    )


NBUF = 4
LOOK = 2
CHA = 64
NCHA = EPT // CHA
HALF = 40
FAST_CID = 1
NF_CHUNKS = 240
NS_CHUNKS = 80


@functools.partial(
    pl.kernel,
    out_type=jax.ShapeDtypeStruct((NC * NPAD, D), jnp.float32),
    mesh=_MESH,
    scratch_types=[
        pltpu.VMEM((HALF, CHA), jnp.int32),
        pltpu.VMEM((HALF, CHA), jnp.int32),
        pltpu.VMEM((NBUF, CHA, D), jnp.float32),
        pltpu.VMEM_SHARED((NPAD, D), jnp.float32),
        pltpu.SemaphoreType.DMA((NBUF,)),
        pltpu.SemaphoreType.DMA((NBUF,)),
    ],
)
def _sc_aggregate(h_hbm, src_hbm, dst_hbm, zrow_hbm, out_hbm,
                  src_v, dst_v, buf_v, acc_sh, gsem, ssem):
    cid = lax.axis_index("c")
    sid = lax.axis_index("s")
    fast = cid == FAST_CID
    nparts = jnp.where(fast, NF_CHUNKS // HALF, NS_CHUNKS // HALF)
    tbase = jnp.where(fast, sid * NF_CHUNKS,
                      NS * NF_CHUNKS + sid * NS_CHUNKS)
    pltpu.sync_copy(zrow_hbm, acc_sh.at[pl.ds(sid * RPT, RPT)])
    plsc.subcore_barrier()

    def gather(j, b):
        pltpu.async_copy(h_hbm.at[src_v.at[j]], buf_v.at[b], gsem.at[b])

    def wait_g(b):
        pltpu.make_async_copy(h_hbm.at[src_v.at[0]], buf_v.at[b],
                              gsem.at[b]).wait()

    def scat(i, b):
        pltpu.async_copy(buf_v.at[b], acc_sh.at[dst_v.at[i]], ssem.at[b],
                         add=True)

    def wait_s(b):
        pltpu.make_async_copy(buf_v.at[b], acc_sh.at[dst_v.at[0]],
                              ssem.at[b]).wait()

    def part_body(p, carry):
        base_chunk = tbase + p * HALF
        pltpu.sync_copy(src_hbm.at[pl.ds(base_chunk, HALF)], src_v)
        pltpu.sync_copy(dst_hbm.at[pl.ds(base_chunk, HALF)], dst_v)

        gather(0, 0)
        gather(1, 1)
        for b in range(NBUF):
            wait_g(b)
            scat(b, b)
            bj = (b + LOOK) % NBUF
            if b >= NBUF - LOOK:
                wait_s(bj)
            gather(b + LOOK, bj)

        def round_body(r, carry):
            for b in range(NBUF):
                i = r * NBUF + b
                wait_g(b)
                scat(i, b)
                bj = (b + LOOK) % NBUF
                wait_s(bj)
                gather(i + LOOK, bj)
            return carry

        lax.fori_loop(1, HALF // NBUF - 1, round_body, 0)

        tail = HALF - NBUF
        for b in range(NBUF):
            wait_g(b)
            scat(tail + b, b)
            if b < LOOK:
                bj = (b + LOOK) % NBUF
                wait_s(bj)
                gather(tail + b + LOOK, bj)
        for b in range(NBUF):
            wait_s(b)
        return carry

    lax.fori_loop(0, nparts, part_body, 0)

    plsc.subcore_barrier()
    pltpu.sync_copy(
        acc_sh.at[pl.ds(sid * RPT, RPT)],
        out_hbm.at[pl.ds(cid * NPAD + sid * RPT, RPT)],
    )


_BLK = 128
_NBLK = NPAD // _BLK


def _scale_body(x_ref, d0_ref, d1_ref, h_ref, inv_ref):
    i = pl.program_id(0)
    deg = d0_ref[:, 0:1] + d1_ref[:, 0:1]
    deg = jnp.maximum(deg, 1.0)
    row = i * _BLK + lax.broadcasted_iota(jnp.int32, (_BLK, 1), 0)
    inv = jnp.where(row < N, 1.0 / deg, 0.0)
    h_ref[...] = x_ref[...] * inv
    inv_ref[...] = inv


def _tc_scale(x_pad, degp):
    return pl.pallas_call(
        _scale_body,
        grid=(_NBLK,),
        in_specs=[
            pl.BlockSpec((_BLK, D), lambda i: (i, 0)),
            pl.BlockSpec((_BLK, D), lambda i: (i, 0)),
            pl.BlockSpec((_BLK, D), lambda i: (i + _NBLK, 0)),
        ],
        out_specs=[
            pl.BlockSpec((_BLK, D), lambda i: (i, 0)),
            pl.BlockSpec((_BLK, 1), lambda i: (i, 0)),
        ],
        out_shape=[
            jax.ShapeDtypeStruct((NPAD, D), jnp.float32),
            jax.ShapeDtypeStruct((NPAD, 1), jnp.float32),
        ],
    )(x_pad, degp, degp)


def _mm_body(scale_out, p0_ref, p1_ref, w_ref, b_ref, a_ref, inv_ref, o_ref):
    agg = p0_ref[...] + p1_ref[...]
    z = jnp.dot(agg, w_ref[...], preferred_element_type=jnp.float32)
    z = z + b_ref[...]
    z = jnp.where(z >= 0.0, z, a_ref[0, 0] * z)
    if scale_out:
        z = z * inv_ref[...]
    o_ref[...] = z


def _tc_matmul(partials, w, b, a, inv, scale_out):
    n_out = NPAD if scale_out else N
    grid = pl.cdiv(n_out, _BLK)
    return pl.pallas_call(
        functools.partial(_mm_body, scale_out),
        grid=(grid,),
        in_specs=[
            pl.BlockSpec((_BLK, D), lambda i: (i, 0)),
            pl.BlockSpec((_BLK, D), lambda i: (i + _NBLK, 0)),
            pl.BlockSpec((D, D), lambda i: (0, 0)),
            pl.BlockSpec((1, D), lambda i: (0, 0)),
            pl.BlockSpec((1, 1), lambda i: (0, 0)),
            pl.BlockSpec((_BLK, 1), lambda i: (i, 0)),
        ],
        out_specs=pl.BlockSpec((_BLK, D), lambda i: (i, 0)),
        out_shape=jax.ShapeDtypeStruct((n_out, D), jnp.float32),
    )(partials, partials, w, b, a, inv)


def kernel(x, edge_index, W0, b0, W1, b1, a):
    src = edge_index[0].astype(jnp.int32)
    dst = edge_index[1].astype(jnp.int32)
    pad = jnp.full((NE_PAD - NE,), N, dtype=jnp.int32)
    src_flat = jnp.concatenate([src, pad])
    dst_flat = jnp.concatenate([dst, pad])
    src_deg = src_flat.reshape(NW * NCHUNK, CH)
    src_p = src_flat.reshape(NW * NCHA, CHA)
    dst_p = dst_flat.reshape(NW * NCHA, CHA)
    x_pad = jnp.concatenate(
        [x.astype(jnp.float32), jnp.zeros((NPAD - N, D), jnp.float32)]
    )
    ones_rows = jnp.ones((CH, D), jnp.float32)
    zrow = jnp.zeros((RPT, D), jnp.float32)
    b0r = b0.reshape(1, D).astype(jnp.float32)
    b1r = b1.reshape(1, D).astype(jnp.float32)
    ar = a.reshape(1, 1).astype(jnp.float32)

    degp = _sc_degree(src_deg, zrow, ones_rows)
    h0, inv = _tc_scale(x_pad, degp)
    p0 = _sc_aggregate(h0, src_p, dst_p, zrow)
    h1 = _tc_matmul(p0, W0.astype(jnp.float32), b0r, ar, inv, True)
    p1 = _sc_aggregate(h1, src_p, dst_p, zrow)
    z2 = _tc_matmul(p1, W1.astype(jnp.float32), b1r, ar, inv, False)
    return z2


# speedup vs baseline: 1.0440x; 1.0440x over previous
import functools

import jax
import jax.numpy as jnp
from jax import lax
from jax.experimental import pallas as pl
from jax.experimental.pallas import tpu as pltpu
from jax.experimental.pallas import tpu_sc as plsc

N = 10000
NE = 320000
D = 128

NC, NS = 2, 16
NW = NC * NS
NPAD = 10240
RPT = NPAD // NS
NE_PAD = 327680
EPT = NE_PAD // NW
CH = 128
NCHUNK = EPT // CH

_MESH = plsc.VectorSubcoreMesh(
    core_axis_name="c", subcore_axis_name="s", num_cores=NC, num_subcores=NS
)


@functools.partial(
    pl.kernel,
    out_type=jax.ShapeDtypeStruct((NC * NPAD, D), jnp.float32),
    mesh=_MESH,
    scratch_types=[
        pltpu.VMEM((NCHUNK, CH), jnp.int32),
        pltpu.VMEM((CH, D), jnp.float32),
        pltpu.VMEM_SHARED((NPAD, D), jnp.float32),
        pltpu.SemaphoreType.DMA,
    ],
)
def _sc_degree(src_hbm, zdeg_hbm, ones_hbm, out_hbm, idx_v, ones_v, acc_sh, sem):
    cid = lax.axis_index("c")
    sid = lax.axis_index("s")
    wid = sid * NC + cid
    pltpu.sync_copy(src_hbm.at[pl.ds(wid * NCHUNK, NCHUNK)], idx_v)
    pltpu.sync_copy(ones_hbm, ones_v)
    pltpu.sync_copy(zdeg_hbm, acc_sh.at[pl.ds(sid * RPT, RPT)])
    plsc.subcore_barrier()

    def fire(ci, carry):
        pltpu.async_copy(ones_v, acc_sh.at[idx_v.at[ci]], sem, add=True)
        return carry

    lax.fori_loop(0, NCHUNK, fire, 0)

    def drain(ci, carry):
        pltpu.make_async_copy(ones_v, acc_sh.at[idx_v.at[0]], sem).wait()
        return carry

    lax.fori_loop(0, NCHUNK, drain, 0)
    plsc.subcore_barrier()
    pltpu.sync_copy(
        acc_sh.at[pl.ds(sid * RPT, RPT)],
        out_hbm.at[pl.ds(cid * NPAD + sid * RPT, RPT)],
    )


NBUF = 4
LOOK = 2
CHA = 64
NCHA = EPT // CHA
HALF = 40
FAST_CID = 0
NF_CHUNKS = 240
NS_CHUNKS = 80


@functools.partial(
    pl.kernel,
    out_type=jax.ShapeDtypeStruct((NC * NPAD, D), jnp.float32),
    mesh=_MESH,
    scratch_types=[
        pltpu.VMEM((HALF, CHA), jnp.int32),
        pltpu.VMEM((HALF, CHA), jnp.int32),
        pltpu.VMEM((NBUF, CHA, D), jnp.float32),
        pltpu.VMEM_SHARED((NPAD, D), jnp.float32),
        pltpu.SemaphoreType.DMA((NBUF,)),
        pltpu.SemaphoreType.DMA((NBUF,)),
    ],
)
def _sc_aggregate(h_hbm, src_hbm, dst_hbm, zrow_hbm, out_hbm,
                  src_v, dst_v, buf_v, acc_sh, gsem, ssem):
    cid = lax.axis_index("c")
    sid = lax.axis_index("s")
    fast = cid == FAST_CID
    nparts = jnp.where(fast, NF_CHUNKS // HALF, NS_CHUNKS // HALF)
    tbase = jnp.where(fast, sid * NF_CHUNKS,
                      NS * NF_CHUNKS + sid * NS_CHUNKS)
    pltpu.sync_copy(zrow_hbm, acc_sh.at[pl.ds(sid * RPT, RPT)])
    plsc.subcore_barrier()

    def gather(j, b):
        pltpu.async_copy(h_hbm.at[src_v.at[j]], buf_v.at[b], gsem.at[b])

    def wait_g(b):
        pltpu.make_async_copy(h_hbm.at[src_v.at[0]], buf_v.at[b],
                              gsem.at[b]).wait()

    def scat(i, b):
        pltpu.async_copy(buf_v.at[b], acc_sh.at[dst_v.at[i]], ssem.at[b],
                         add=True)

    def wait_s(b):
        pltpu.make_async_copy(buf_v.at[b], acc_sh.at[dst_v.at[0]],
                              ssem.at[b]).wait()

    def part_body(p, carry):
        base_chunk = tbase + p * HALF
        pltpu.sync_copy(src_hbm.at[pl.ds(base_chunk, HALF)], src_v)
        pltpu.sync_copy(dst_hbm.at[pl.ds(base_chunk, HALF)], dst_v)

        gather(0, 0)
        gather(1, 1)
        for b in range(NBUF):
            wait_g(b)
            scat(b, b)
            bj = (b + LOOK) % NBUF
            if b >= NBUF - LOOK:
                wait_s(bj)
            gather(b + LOOK, bj)

        def round_body(r, carry):
            for b in range(NBUF):
                i = r * NBUF + b
                wait_g(b)
                scat(i, b)
                bj = (b + LOOK) % NBUF
                wait_s(bj)
                gather(i + LOOK, bj)
            return carry

        lax.fori_loop(1, HALF // NBUF - 1, round_body, 0)

        tail = HALF - NBUF
        for b in range(NBUF):
            wait_g(b)
            scat(tail + b, b)
            if b < LOOK:
                bj = (b + LOOK) % NBUF
                wait_s(bj)
                gather(tail + b + LOOK, bj)
        for b in range(NBUF):
            wait_s(b)
        return carry

    lax.fori_loop(0, nparts, part_body, 0)

    plsc.subcore_barrier()
    pltpu.sync_copy(
        acc_sh.at[pl.ds(sid * RPT, RPT)],
        out_hbm.at[pl.ds(cid * NPAD + sid * RPT, RPT)],
    )


_BLK = 128
_NBLK = NPAD // _BLK


def _scale_body(x_ref, d0_ref, d1_ref, h_ref, inv_ref):
    i = pl.program_id(0)
    deg = d0_ref[:, 0:1] + d1_ref[:, 0:1]
    deg = jnp.maximum(deg, 1.0)
    row = i * _BLK + lax.broadcasted_iota(jnp.int32, (_BLK, 1), 0)
    inv = jnp.where(row < N, 1.0 / deg, 0.0)
    h_ref[...] = x_ref[...] * inv
    inv_ref[...] = inv


def _tc_scale(x_pad, degp):
    return pl.pallas_call(
        _scale_body,
        grid=(_NBLK,),
        in_specs=[
            pl.BlockSpec((_BLK, D), lambda i: (i, 0)),
            pl.BlockSpec((_BLK, D), lambda i: (i, 0)),
            pl.BlockSpec((_BLK, D), lambda i: (i + _NBLK, 0)),
        ],
        out_specs=[
            pl.BlockSpec((_BLK, D), lambda i: (i, 0)),
            pl.BlockSpec((_BLK, 1), lambda i: (i, 0)),
        ],
        out_shape=[
            jax.ShapeDtypeStruct((NPAD, D), jnp.float32),
            jax.ShapeDtypeStruct((NPAD, 1), jnp.float32),
        ],
    )(x_pad, degp, degp)


def _mm_body(scale_out, p0_ref, p1_ref, w_ref, b_ref, a_ref, inv_ref, o_ref):
    agg = p0_ref[...] + p1_ref[...]
    z = jnp.dot(agg, w_ref[...], preferred_element_type=jnp.float32)
    z = z + b_ref[...]
    z = jnp.where(z >= 0.0, z, a_ref[0, 0] * z)
    if scale_out:
        z = z * inv_ref[...]
    o_ref[...] = z


def _tc_matmul(partials, w, b, a, inv, scale_out):
    n_out = NPAD if scale_out else N
    grid = pl.cdiv(n_out, _BLK)
    return pl.pallas_call(
        functools.partial(_mm_body, scale_out),
        grid=(grid,),
        in_specs=[
            pl.BlockSpec((_BLK, D), lambda i: (i, 0)),
            pl.BlockSpec((_BLK, D), lambda i: (i + _NBLK, 0)),
            pl.BlockSpec((D, D), lambda i: (0, 0)),
            pl.BlockSpec((1, D), lambda i: (0, 0)),
            pl.BlockSpec((1, 1), lambda i: (0, 0)),
            pl.BlockSpec((_BLK, 1), lambda i: (i, 0)),
        ],
        out_specs=pl.BlockSpec((_BLK, D), lambda i: (i, 0)),
        out_shape=jax.ShapeDtypeStruct((n_out, D), jnp.float32),
    )(partials, partials, w, b, a, inv)


def kernel(x, edge_index, W0, b0, W1, b1, a):
    src = edge_index[0].astype(jnp.int32)
    dst = edge_index[1].astype(jnp.int32)
    pad = jnp.full((NE_PAD - NE,), N, dtype=jnp.int32)
    src_flat = jnp.concatenate([src, pad])
    dst_flat = jnp.concatenate([dst, pad])
    src_deg = src_flat.reshape(NW * NCHUNK, CH)
    src_p = src_flat.reshape(NW * NCHA, CHA)
    dst_p = dst_flat.reshape(NW * NCHA, CHA)
    x_pad = jnp.concatenate(
        [x.astype(jnp.float32), jnp.zeros((NPAD - N, D), jnp.float32)]
    )
    ones_rows = jnp.ones((CH, D), jnp.float32)
    zrow = jnp.zeros((RPT, D), jnp.float32)
    b0r = b0.reshape(1, D).astype(jnp.float32)
    b1r = b1.reshape(1, D).astype(jnp.float32)
    ar = a.reshape(1, 1).astype(jnp.float32)

    degp = _sc_degree(src_deg, zrow, ones_rows)
    h0, inv = _tc_scale(x_pad, degp)
    p0 = _sc_aggregate(h0, src_p, dst_p, zrow)
    h1 = _tc_matmul(p0, W0.astype(jnp.float32), b0r, ar, inv, True)
    p1 = _sc_aggregate(h1, src_p, dst_p, zrow)
    z2 = _tc_matmul(p1, W1.astype(jnp.float32), b1r, ar, inv, False)
    return z2


# speedup vs baseline: 1.0944x; 1.0483x over previous
import functools

import jax
import jax.numpy as jnp
from jax import lax
from jax.experimental import pallas as pl
from jax.experimental.pallas import tpu as pltpu
from jax.experimental.pallas import tpu_sc as plsc

N = 10000
NE = 320000
D = 128

NC, NS = 2, 16
NW = NC * NS
NPAD = 10240
RPT = NPAD // NS
NE_PAD = 327680
EPT = NE_PAD // NW
CH = 128
NCHUNK = EPT // CH

_MESH = plsc.VectorSubcoreMesh(
    core_axis_name="c", subcore_axis_name="s", num_cores=NC, num_subcores=NS
)


@functools.partial(
    pl.kernel,
    out_type=jax.ShapeDtypeStruct((NC * NPAD, D), jnp.float32),
    mesh=_MESH,
    scratch_types=[
        pltpu.VMEM((NCHUNK, CH), jnp.int32),
        pltpu.VMEM((CH, D), jnp.float32),
        pltpu.VMEM_SHARED((NPAD, D), jnp.float32),
        pltpu.SemaphoreType.DMA,
    ],
)
def _sc_degree(src_hbm, zdeg_hbm, ones_hbm, out_hbm, idx_v, ones_v, acc_sh, sem):
    cid = lax.axis_index("c")
    sid = lax.axis_index("s")
    wid = sid * NC + cid
    pltpu.sync_copy(src_hbm.at[pl.ds(wid * NCHUNK, NCHUNK)], idx_v)
    pltpu.sync_copy(ones_hbm, ones_v)
    pltpu.sync_copy(zdeg_hbm, acc_sh.at[pl.ds(sid * RPT, RPT)])
    plsc.subcore_barrier()

    def fire(ci, carry):
        pltpu.async_copy(ones_v, acc_sh.at[idx_v.at[ci]], sem, add=True)
        return carry

    lax.fori_loop(0, NCHUNK, fire, 0)

    def drain(ci, carry):
        pltpu.make_async_copy(ones_v, acc_sh.at[idx_v.at[0]], sem).wait()
        return carry

    lax.fori_loop(0, NCHUNK, drain, 0)
    plsc.subcore_barrier()
    pltpu.sync_copy(
        acc_sh.at[pl.ds(sid * RPT, RPT)],
        out_hbm.at[pl.ds(cid * NPAD + sid * RPT, RPT)],
    )


NBUF = 8
LOOK = 4
CHA = 32
NCHA = EPT // CHA
HALF = 40
FAST_CID = 0
NF_CHUNKS = 480
NS_CHUNKS = 160


@functools.partial(
    pl.kernel,
    out_type=jax.ShapeDtypeStruct((NC * NPAD, D), jnp.float32),
    mesh=_MESH,
    scratch_types=[
        pltpu.VMEM((HALF, CHA), jnp.int32),
        pltpu.VMEM((HALF, CHA), jnp.int32),
        pltpu.VMEM((NBUF, CHA, D), jnp.float32),
        pltpu.VMEM_SHARED((NPAD, D), jnp.float32),
        pltpu.SemaphoreType.DMA((NBUF,)),
        pltpu.SemaphoreType.DMA((NBUF,)),
    ],
)
def _sc_aggregate(h_hbm, src_hbm, dst_hbm, zrow_hbm, out_hbm,
                  src_v, dst_v, buf_v, acc_sh, gsem, ssem):
    cid = lax.axis_index("c")
    sid = lax.axis_index("s")
    fast = cid == FAST_CID
    nparts = jnp.where(fast, NF_CHUNKS // HALF, NS_CHUNKS // HALF)
    tbase = jnp.where(fast, sid * NF_CHUNKS,
                      NS * NF_CHUNKS + sid * NS_CHUNKS)
    pltpu.sync_copy(zrow_hbm, acc_sh.at[pl.ds(sid * RPT, RPT)])
    plsc.subcore_barrier()

    def gather(j, b):
        pltpu.async_copy(h_hbm.at[src_v.at[j]], buf_v.at[b], gsem.at[b])

    def wait_g(b):
        pltpu.make_async_copy(h_hbm.at[src_v.at[0]], buf_v.at[b],
                              gsem.at[b]).wait()

    def scat(i, b):
        pltpu.async_copy(buf_v.at[b], acc_sh.at[dst_v.at[i]], ssem.at[b],
                         add=True)

    def wait_s(b):
        pltpu.make_async_copy(buf_v.at[b], acc_sh.at[dst_v.at[0]],
                              ssem.at[b]).wait()

    def part_body(p, carry):
        base_chunk = tbase + p * HALF
        pltpu.sync_copy(src_hbm.at[pl.ds(base_chunk, HALF)], src_v)
        pltpu.sync_copy(dst_hbm.at[pl.ds(base_chunk, HALF)], dst_v)

        for k in range(LOOK):
            gather(k, k)
        for b in range(NBUF):
            wait_g(b)
            scat(b, b)
            bj = (b + LOOK) % NBUF
            if b >= NBUF - LOOK:
                wait_s(bj)
            gather(b + LOOK, bj)

        def round_body(r, carry):
            for b in range(NBUF):
                i = r * NBUF + b
                wait_g(b)
                scat(i, b)
                bj = (b + LOOK) % NBUF
                wait_s(bj)
                gather(i + LOOK, bj)
            return carry

        lax.fori_loop(1, HALF // NBUF - 1, round_body, 0)

        tail = HALF - NBUF
        for b in range(NBUF):
            wait_g(b)
            scat(tail + b, b)
            if b < LOOK:
                bj = (b + LOOK) % NBUF
                wait_s(bj)
                gather(tail + b + LOOK, bj)
        for b in range(NBUF):
            wait_s(b)
        return carry

    lax.fori_loop(0, nparts, part_body, 0)

    plsc.subcore_barrier()
    pltpu.sync_copy(
        acc_sh.at[pl.ds(sid * RPT, RPT)],
        out_hbm.at[pl.ds(cid * NPAD + sid * RPT, RPT)],
    )


_BLK = 128
_NBLK = NPAD // _BLK


def _scale_body(x_ref, d0_ref, d1_ref, h_ref, inv_ref):
    i = pl.program_id(0)
    deg = d0_ref[:, 0:1] + d1_ref[:, 0:1]
    deg = jnp.maximum(deg, 1.0)
    row = i * _BLK + lax.broadcasted_iota(jnp.int32, (_BLK, 1), 0)
    inv = jnp.where(row < N, 1.0 / deg, 0.0)
    h_ref[...] = x_ref[...] * inv
    inv_ref[...] = inv


def _tc_scale(x_pad, degp):
    return pl.pallas_call(
        _scale_body,
        grid=(_NBLK,),
        in_specs=[
            pl.BlockSpec((_BLK, D), lambda i: (i, 0)),
            pl.BlockSpec((_BLK, D), lambda i: (i, 0)),
            pl.BlockSpec((_BLK, D), lambda i: (i + _NBLK, 0)),
        ],
        out_specs=[
            pl.BlockSpec((_BLK, D), lambda i: (i, 0)),
            pl.BlockSpec((_BLK, 1), lambda i: (i, 0)),
        ],
        out_shape=[
            jax.ShapeDtypeStruct((NPAD, D), jnp.float32),
            jax.ShapeDtypeStruct((NPAD, 1), jnp.float32),
        ],
    )(x_pad, degp, degp)


def _mm_body(scale_out, p0_ref, p1_ref, w_ref, b_ref, a_ref, inv_ref, o_ref):
    agg = p0_ref[...] + p1_ref[...]
    z = jnp.dot(agg, w_ref[...], preferred_element_type=jnp.float32)
    z = z + b_ref[...]
    z = jnp.where(z >= 0.0, z, a_ref[0, 0] * z)
    if scale_out:
        z = z * inv_ref[...]
    o_ref[...] = z


def _tc_matmul(partials, w, b, a, inv, scale_out):
    n_out = NPAD if scale_out else N
    grid = pl.cdiv(n_out, _BLK)
    return pl.pallas_call(
        functools.partial(_mm_body, scale_out),
        grid=(grid,),
        in_specs=[
            pl.BlockSpec((_BLK, D), lambda i: (i, 0)),
            pl.BlockSpec((_BLK, D), lambda i: (i + _NBLK, 0)),
            pl.BlockSpec((D, D), lambda i: (0, 0)),
            pl.BlockSpec((1, D), lambda i: (0, 0)),
            pl.BlockSpec((1, 1), lambda i: (0, 0)),
            pl.BlockSpec((_BLK, 1), lambda i: (i, 0)),
        ],
        out_specs=pl.BlockSpec((_BLK, D), lambda i: (i, 0)),
        out_shape=jax.ShapeDtypeStruct((n_out, D), jnp.float32),
    )(partials, partials, w, b, a, inv)


def kernel(x, edge_index, W0, b0, W1, b1, a):
    src = edge_index[0].astype(jnp.int32)
    dst = edge_index[1].astype(jnp.int32)
    pad = jnp.full((NE_PAD - NE,), N, dtype=jnp.int32)
    src_flat = jnp.concatenate([src, pad])
    dst_flat = jnp.concatenate([dst, pad])
    src_deg = src_flat.reshape(NW * NCHUNK, CH)
    src_p = src_flat.reshape(NW * NCHA, CHA)
    dst_p = dst_flat.reshape(NW * NCHA, CHA)
    x_pad = jnp.concatenate(
        [x.astype(jnp.float32), jnp.zeros((NPAD - N, D), jnp.float32)]
    )
    ones_rows = jnp.ones((CH, D), jnp.float32)
    zrow = jnp.zeros((RPT, D), jnp.float32)
    b0r = b0.reshape(1, D).astype(jnp.float32)
    b1r = b1.reshape(1, D).astype(jnp.float32)
    ar = a.reshape(1, 1).astype(jnp.float32)

    degp = _sc_degree(src_deg, zrow, ones_rows)
    h0, inv = _tc_scale(x_pad, degp)
    p0 = _sc_aggregate(h0, src_p, dst_p, zrow)
    h1 = _tc_matmul(p0, W0.astype(jnp.float32), b0r, ar, inv, True)
    p1 = _sc_aggregate(h1, src_p, dst_p, zrow)
    z2 = _tc_matmul(p1, W1.astype(jnp.float32), b1r, ar, inv, False)
    return z2
